# baseline (speedup 1.0000x reference)
import functools

import jax
import jax.numpy as jnp
from jax import lax
from jax.experimental import pallas as pl
from jax.experimental.pallas import tpu as pltpu
from jax.experimental.pallas import tpu_sc as plsc

N = 10000
E = 320000
D = 128
DE = 16
L = 3

NC = 2
NS = 16
NW = NC * NS
CHUNK = 128
EPT = E // NW
NCH = 80
EPT_PAD = NCH * CHUNK
ACC_ROWS = EPT_PAD
PAD_DST = N
INIT_ROWS = ACC_ROWS // NS
FULL = EPT // CHUNK
TAIL = EPT - FULL * CHUNK


@functools.cache
def _mesh():
    return plsc.VectorSubcoreMesh(core_axis_name="c", subcore_axis_name="s",
                                  num_cores=NC, num_subcores=NS)


HALF = NCH // 2


def _spmv_body(x_hbm, srcs_hbm, dsts_hbm, zz_hbm, part_hbm,
               rows_a, rows_b, src_v, dst_v, acc, sem_a, sem_b):
    c = lax.axis_index("c")
    s = lax.axis_index("s")
    wid = c * NS + s

    pltpu.sync_copy(zz_hbm.at[pl.ds(s * INIT_ROWS, INIT_ROWS)],
                    acc.at[pl.ds(s * INIT_ROWS, INIT_ROWS)])
    plsc.subcore_barrier()

    for p in range(2):
        pltpu.sync_copy(srcs_hbm.at[wid, pl.ds(p * HALF, HALF)], src_v)
        pltpu.sync_copy(dsts_hbm.at[wid, pl.ds(p * HALF, HALF)], dst_v)
        pltpu.async_copy(x_hbm.at[src_v.at[0]], rows_a, sem_a)

        def pair(k, carry):
            j0 = 2 * k
            pltpu.make_async_copy(x_hbm.at[src_v.at[j0]], rows_a, sem_a).wait()
            pltpu.async_copy(x_hbm.at[src_v.at[j0 + 1]], rows_b, sem_b)
            pltpu.sync_copy(rows_a, acc.at[dst_v.at[j0]], add=True)
            pltpu.make_async_copy(x_hbm.at[src_v.at[j0 + 1]], rows_b,
                                  sem_b).wait()

            @pl.when(j0 + 2 < HALF)
            def _():
                pltpu.async_copy(x_hbm.at[src_v.at[j0 + 2]], rows_a, sem_a)

            pltpu.sync_copy(rows_b, acc.at[dst_v.at[j0 + 1]], add=True)
            return carry

        lax.fori_loop(0, HALF // 2, pair, 0)
    plsc.subcore_barrier()

    pltpu.sync_copy(acc.at[pl.ds(s * INIT_ROWS, INIT_ROWS)],
                    part_hbm.at[c, pl.ds(s * INIT_ROWS, INIT_ROWS)])


@functools.cache
def _sc_spmv():
    return pl.kernel(
        _spmv_body,
        out_type=jax.ShapeDtypeStruct((NC, ACC_ROWS, D), jnp.float32),
        mesh=_mesh(),
        scratch_types=[
            pltpu.VMEM((CHUNK, D), jnp.float32),
            pltpu.VMEM((CHUNK, D), jnp.float32),
            pltpu.VMEM((HALF, CHUNK), jnp.int32),
            pltpu.VMEM((HALF, CHUNK), jnp.int32),
            pltpu.VMEM_SHARED((ACC_ROWS, D), jnp.float32),
            pltpu.SemaphoreType.DMA,
            pltpu.SemaphoreType.DMA,
        ],
    )


def _eagg_body(ea_hbm, dsts_hbm, zz2_hbm, eagg_hbm,
               ea_a, ea_b, dst_v, acc2, sem_a, sem_b):
    c = lax.axis_index("c")
    s = lax.axis_index("s")
    wid = c * NS + s
    base = wid * EPT

    pltpu.sync_copy(zz2_hbm.at[pl.ds(s * INIT_ROWS, INIT_ROWS)],
                    acc2.at[pl.ds(s * INIT_ROWS, INIT_ROWS)])
    pltpu.sync_copy(dsts_hbm.at[wid], dst_v)
    plsc.subcore_barrier()

    pltpu.async_copy(ea_hbm.at[pl.ds(base, CHUNK)], ea_a, sem_a)

    def pair(k, carry):
        j0 = 2 * k
        pltpu.make_async_copy(ea_hbm.at[pl.ds(base, CHUNK)], ea_a, sem_a).wait()
        pltpu.async_copy(ea_hbm.at[pl.ds(base + (j0 + 1) * CHUNK, CHUNK)],
                         ea_b, sem_b)
        pltpu.sync_copy(ea_a, acc2.at[dst_v.at[j0]], add=True)
        pltpu.make_async_copy(ea_hbm.at[pl.ds(base, CHUNK)], ea_b, sem_b).wait()

        @pl.when(j0 + 2 < FULL)
        def _():
            pltpu.async_copy(ea_hbm.at[pl.ds(base + (j0 + 2) * CHUNK, CHUNK)],
                             ea_a, sem_a)

        pltpu.sync_copy(ea_b, acc2.at[dst_v.at[j0 + 1]], add=True)
        return carry

    lax.fori_loop(0, FULL // 2, pair, 0)

    pltpu.sync_copy(ea_hbm.at[pl.ds(base + FULL * CHUNK, TAIL)],
                    ea_a.at[pl.ds(0, TAIL)])
    pltpu.sync_copy(ea_a, acc2.at[dst_v.at[FULL]], add=True)
    plsc.subcore_barrier()

    pltpu.sync_copy(acc2.at[pl.ds(s * INIT_ROWS, INIT_ROWS)],
                    eagg_hbm.at[c, pl.ds(s * INIT_ROWS, INIT_ROWS)])


@functools.cache
def _sc_eagg():
    return pl.kernel(
        _eagg_body,
        out_type=jax.ShapeDtypeStruct((NC, ACC_ROWS, DE), jnp.float32),
        mesh=_mesh(),
        scratch_types=[
            pltpu.VMEM((CHUNK, DE), jnp.float32),
            pltpu.VMEM((CHUNK, DE), jnp.float32),
            pltpu.VMEM((NCH, CHUNK), jnp.int32),
            pltpu.VMEM_SHARED((ACC_ROWS, DE), jnp.float32),
            pltpu.SemaphoreType.DMA,
            pltpu.SemaphoreType.DMA,
        ],
        compiler_params=pltpu.CompilerParams(use_tc_tiling_on_sc=False),
    )


def _tc_layer(part_ref, eaggp_ref, wes_ref, w_ref, b_ref, g_ref, be_ref,
              out_ref):
    s = part_ref[0, :N] + part_ref[1, :N]
    eagg = eaggp_ref[0, :N] + eaggp_ref[1, :N]
    agg = s + jnp.dot(eagg, wes_ref[...], preferred_element_type=jnp.float32)
    h = jnp.dot(agg, w_ref[...], preferred_element_type=jnp.float32)
    h = h + b_ref[...]
    mean = jnp.mean(h, axis=0, keepdims=True)
    var = jnp.mean((h - mean) ** 2, axis=0, keepdims=True)
    h = (h - mean) * jax.lax.rsqrt(var + 1e-5) * g_ref[...] + be_ref[...]
    out_ref[...] = jnp.maximum(h, 0.0)


_tc_call = pl.pallas_call(
    _tc_layer,
    out_shape=jax.ShapeDtypeStruct((N, D), jnp.float32),
)


def kernel(x, edge_index, edge_attr, batch, Ws, bs, Wes, gammas, betas):
    del batch
    src = edge_index[0].astype(jnp.int32)
    dst = edge_index[1].astype(jnp.int32)

    srcs = jnp.pad(src.reshape(NW, EPT), ((0, 0), (0, EPT_PAD - EPT)))
    srcs = srcs.reshape(NW, NCH, CHUNK)
    dsts = jnp.pad(dst.reshape(NW, EPT), ((0, 0), (0, EPT_PAD - EPT)),
                   constant_values=PAD_DST).reshape(NW, NCH, CHUNK)
    zz = jnp.zeros((ACC_ROWS, D), jnp.float32)
    zz2 = jnp.zeros((ACC_ROWS, DE), jnp.float32)

    x = x.astype(jnp.float32)
    eaggp = _sc_eagg()(edge_attr.astype(jnp.float32), dsts, zz2)
    for l in range(L):
        part = _sc_spmv()(x, srcs, dsts, zz)
        x = _tc_call(part, eaggp, Wes[l], Ws[l], bs[l].reshape(1, D),
                     gammas[l].reshape(1, D), betas[l].reshape(1, D))
    return (x, L)
